# baseline (speedup 1.0000x reference)
import functools

import jax
import jax.numpy as jnp
from jax import lax
from jax.experimental import pallas as pl
from jax.experimental.pallas import tpu as pltpu
from jax.experimental.pallas import tpu_sc as plsc

CHUNK = 128
NBUF = 5


def _sc_gather(tokens2d, table):
    n_chunks, chunk = tokens2d.shape
    assert chunk == CHUNK
    d = table.shape[1]

    info = plsc.get_sparse_core_info()
    nc, ns = info.num_cores, info.num_subcores
    nw = nc * ns
    chunks_per_w = n_chunks // nw
    assert chunks_per_w * nw == n_chunks
    assert chunks_per_w % NBUF == 0
    n_outer = chunks_per_w // NBUF
    total_rows = n_chunks * CHUNK

    mesh = plsc.VectorSubcoreMesh(core_axis_name="c", subcore_axis_name="s")
    scratch = [pltpu.VMEM((chunks_per_w, CHUNK), jnp.int32)]
    scratch += [pltpu.VMEM((CHUNK, d), jnp.float32) for _ in range(NBUF)]
    scratch += [pltpu.SemaphoreType.DMA for _ in range(2 * NBUF)]

    @functools.partial(
        pl.kernel,
        mesh=mesh,
        out_type=jax.ShapeDtypeStruct((total_rows, d), jnp.float32),
        scratch_types=scratch,
    )
    def gather_kernel(tokens_hbm, table_hbm, out_hbm, *refs):
        idx_v = refs[0]
        bufs = refs[1:1 + NBUF]
        gsems = refs[1 + NBUF:1 + 2 * NBUF]
        wsems = refs[1 + 2 * NBUF:1 + 3 * NBUF]

        wid = lax.axis_index("s") * nc + lax.axis_index("c")
        chunk0 = wid * chunks_per_w

        pltpu.sync_copy(tokens_hbm.at[pl.ds(chunk0, chunks_per_w)], idx_v)

        def g_copy(j, b):
            return pltpu.make_async_copy(
                table_hbm.at[idx_v.at[j]], bufs[b], gsems[b])

        def w_copy(j, b):
            return pltpu.make_async_copy(
                bufs[b],
                out_hbm.at[pl.ds((chunk0 + j) * CHUNK, CHUNK)],
                wsems[b])

        for b in range(NBUF):
            g_copy(b, b).start()

        def body(k, carry):
            j0 = k * NBUF
            for b in range(NBUF):
                g_copy(j0 + b, b).wait()
                w_copy(j0 + b, b).start()
            for b in range(NBUF):
                w_copy(j0 + b, b).wait()
                g_copy(j0 + NBUF + b, b).start()
            return carry

        lax.fori_loop(0, n_outer - 1, body, 0)

        j0 = (n_outer - 1) * NBUF
        for b in range(NBUF):
            g_copy(j0 + b, b).wait()
            w_copy(j0 + b, b).start()
        for b in range(NBUF):
            w_copy(j0 + b, b).wait()

    return gather_kernel(tokens2d, table)


TBLK = 32768


def _tc_transpose_pad(table_t):
    d, vocab = table_t.shape
    n_blk = (vocab + TBLK - 1) // TBLK

    def body(t_ref, o_ref):
        o_ref[:, :d] = t_ref[...].T

    return pl.pallas_call(
        body,
        grid=(n_blk,),
        in_specs=[pl.BlockSpec((d, TBLK), lambda i: (0, i))],
        out_specs=pl.BlockSpec((TBLK, CHUNK), lambda i: (i, 0)),
        out_shape=jax.ShapeDtypeStruct((vocab, CHUNK), jnp.float32),
    )(table_t)


def kernel(tokens, embedding_table):
    batch, max_len = tokens.shape
    d = embedding_table.shape[1]
    table128 = _tc_transpose_pad(embedding_table.T)
    flat = tokens.reshape(batch * max_len // CHUNK, CHUNK)
    out128 = _sc_gather(flat, table128)
    return out128.reshape(batch, max_len, CHUNK)[:, :, :d]
